# baseline (speedup 1.0000x reference)
import functools

import jax
import jax.numpy as jnp
from jax import lax
from jax.experimental import pallas as pl
from jax.experimental.pallas import tpu as pltpu
from jax.experimental.pallas import tpu_sc as plsc

E = 8
TOP_K = 2
LANES = 128
T = 2048
H = 1024
FFN = 2048
TILE = 256
NP = T * TOP_K + E * TILE
NT = NP // TILE
NCHUNK = 4
NT_HALF = NT // NCHUNK
NP_HALF = NP // NCHUNK
NWORK = 32


def _router_body(x_ref, gate_ref, logits_ref, pos1_ref, pos2_ref,
                 wn1_ref, wn2_ref, pf_ref):
    x = x_ref[...]
    gate = gate_ref[...]
    logits = lax.dot_general(x, gate, (((1,), (1,)), ((), ())),
                             preferred_element_type=jnp.float32)
    lane = lax.broadcasted_iota(jnp.int32, (T, LANES), 1)
    neg = jnp.float32(-1e30)
    masked = jnp.where(lane < E, logits, neg)
    m = jnp.max(masked, axis=1, keepdims=True)
    ex = jnp.exp(masked - m)
    p = ex / jnp.sum(ex, axis=1, keepdims=True)
    m1 = jnp.max(p, axis=1, keepdims=True)
    i1 = jnp.min(jnp.where(p == m1, lane, LANES), axis=1, keepdims=True)
    p_rest = jnp.where(lane == i1, jnp.float32(-1.0), p)
    m2 = jnp.max(p_rest, axis=1, keepdims=True)
    i2 = jnp.min(jnp.where(p_rest == m2, lane, LANES), axis=1, keepdims=True)
    denom = m1 + m2
    sel1 = lane == i1
    sel2 = lane == i2
    mask = jnp.where(sel1 | sel2, jnp.float32(1.0), 0.0)
    r_io = lax.broadcasted_iota(jnp.int32, (T, T), 0)
    c_io = lax.broadcasted_iota(jnp.int32, (T, T), 1)
    tril = jnp.where(r_io > c_io, jnp.float32(1.0), 0.0)
    cum = lax.dot_general(tril, mask, (((1,), (0,)), ((), ())),
                          preferred_element_type=jnp.float32)
    counts = jnp.sum(mask, axis=0, keepdims=True)
    padded = jnp.ceil(counts / TILE) * TILE
    ri = lax.broadcasted_iota(jnp.int32, (LANES, LANES), 0)
    ci = lax.broadcasted_iota(jnp.int32, (LANES, LANES), 1)
    upper = jnp.where(ri < ci, jnp.float32(1.0), 0.0)
    offs = lax.dot_general(padded, upper, (((1,), (0,)), ((), ())),
                           preferred_element_type=jnp.float32)
    ends = offs + padded
    pos = offs + cum
    pos1 = jnp.sum(jnp.where(sel1, pos, 0.0), axis=1, keepdims=True)
    pos2 = jnp.sum(jnp.where(sel2, pos, 0.0), axis=1, keepdims=True)
    tstart = (lax.broadcasted_iota(jnp.int32, (LANES, LANES), 0)
              * TILE).astype(jnp.float32)
    lane2 = lax.broadcasted_iota(jnp.int32, (LANES, LANES), 1)
    hit = jnp.where((tstart >= jnp.broadcast_to(ends, (LANES, LANES)))
                    & (lane2 < E), jnp.float32(1.0), 0.0)
    cnt = jnp.sum(hit, axis=1, keepdims=True).astype(jnp.int32)
    logits_ref[...] = logits[:, :E]
    pos1_ref[...] = pos1[:, 0].astype(jnp.int32)
    pos2_ref[...] = pos2[:, 0].astype(jnp.int32)
    wn1_ref[...] = (m1 / denom)[:, 0]
    wn2_ref[...] = (m2 / denom)[:, 0]
    te = jnp.minimum(cnt, E - 1)[:NT, 0]
    valid = jnp.where(cnt < E, 1, 0).astype(jnp.int32)[:NT, 0]
    pf_ref[0, :] = te
    pf_ref[1, :] = valid


def _router(x, gate_pad):
    return pl.pallas_call(
        _router_body,
        out_shape=(
            jax.ShapeDtypeStruct((T, E), jnp.float32),
            jax.ShapeDtypeStruct((T,), jnp.int32),
            jax.ShapeDtypeStruct((T,), jnp.int32),
            jax.ShapeDtypeStruct((T,), jnp.float32),
            jax.ShapeDtypeStruct((T,), jnp.float32),
            jax.ShapeDtypeStruct((2, NT), jnp.int32),
        ),
    )(x, gate_pad)


def _sc_mesh():
    return plsc.VectorSubcoreMesh(core_axis_name="c", subcore_axis_name="s",
                                  num_cores=2, num_subcores=16)


@functools.lru_cache(maxsize=None)
def _sc_meta_kernel():
    return functools.partial(
        pl.kernel,
        mesh=_sc_mesh(),
        out_type=(
            jax.ShapeDtypeStruct((NP,), jnp.int32),
            jax.ShapeDtypeStruct((NP,), jnp.float32),
        ),
        scratch_types=[
            pltpu.VMEM((T,), jnp.int32),
            pltpu.VMEM((T,), jnp.int32),
            pltpu.VMEM((T,), jnp.float32),
            pltpu.VMEM((T,), jnp.float32),
            pltpu.VMEM((NP,), jnp.int32),
            pltpu.VMEM((NP,), jnp.float32),
        ],
        compiler_params=pltpu.CompilerParams(needs_layout_passes=False),
    )(_sc_meta_body)


def _sc_meta_body(p1_hbm, p2_hbm, a1_hbm, a2_hbm, tok_hbm, wgt_hbm,
                  p1_v, p2_v, a1_v, a2_v, tok_v, wgt_v):
    wid = lax.axis_index("s") * 2 + lax.axis_index("c")

    @pl.when(wid == 0)
    def _():
        pltpu.sync_copy(p1_hbm, p1_v)
        pltpu.sync_copy(p2_hbm, p2_v)
        pltpu.sync_copy(a1_hbm, a1_v)
        pltpu.sync_copy(a2_hbm, a2_v)

        def init(i, carry):
            tok_v[pl.ds(i * 16, 16)] = jnp.zeros((16,), jnp.int32)
            wgt_v[pl.ds(i * 16, 16)] = jnp.zeros((16,), jnp.float32)
            return carry

        lax.fori_loop(0, NP // 16, init, 0)

        def scat(i, carry):
            sl = pl.ds(i * 16, 16)
            tvec = lax.iota(jnp.int32, 16) + i * 16
            plsc.store_scatter(tok_v, [p1_v[sl]], tvec)
            plsc.store_scatter(wgt_v, [p1_v[sl]], a1_v[sl])
            plsc.store_scatter(tok_v, [p2_v[sl]], tvec)
            plsc.store_scatter(wgt_v, [p2_v[sl]], a2_v[sl])
            return carry

        lax.fori_loop(0, T // 16, scat, 0)
        pltpu.sync_copy(tok_v, tok_hbm)
        pltpu.sync_copy(wgt_v, wgt_hbm)


def _sc_meta(p1, p2, a1, a2):
    return _sc_meta_kernel()(p1, p2, a1, a2)


_SLOTS_PER = NP_HALF // NWORK
_GCH = _SLOTS_PER // 2


@functools.lru_cache(maxsize=None)
def _sc_gather_kernel(lo):
    body = functools.partial(_sc_gather_body, lo)
    return functools.partial(
        pl.kernel,
        mesh=_sc_mesh(),
        out_type=jax.ShapeDtypeStruct((NP_HALF, H), jnp.float32),
        scratch_types=[
            pltpu.VMEM((_SLOTS_PER,), jnp.int32),
            pltpu.VMEM((_GCH, H), jnp.float32),
            pltpu.VMEM((_GCH, H), jnp.float32),
            pltpu.SemaphoreType.DMA,
            pltpu.SemaphoreType.DMA,
            pltpu.SemaphoreType.DMA,
            pltpu.SemaphoreType.DMA,
        ],
        compiler_params=pltpu.CompilerParams(needs_layout_passes=False),
    )(body)


def _sc_gather_body(lo, x_hbm, tok_hbm, xs_hbm,
                    idx_v, rows_a, rows_b, sem_ga, sem_gb, sem_sa, sem_sb):
    wid = lax.axis_index("s") * 2 + lax.axis_index("c")
    base = wid * _SLOTS_PER
    pltpu.sync_copy(tok_hbm.at[pl.ds(lo + base, _SLOTS_PER)], idx_v)
    bufs = (rows_a, rows_b)
    gsems = (sem_ga, sem_gb)
    ssems = (sem_sa, sem_sb)
    stores = [None, None]
    for c in range(_SLOTS_PER // _GCH):
        b = c % 2
        if stores[b] is not None:
            stores[b].wait()
        pltpu.async_copy(
            x_hbm.at[idx_v.at[pl.ds(c * _GCH, _GCH)]], bufs[b], gsems[b]
        ).wait()
        stores[b] = pltpu.async_copy(
            bufs[b], xs_hbm.at[pl.ds(base + c * _GCH, _GCH)], ssems[b]
        )
    for st in stores:
        if st is not None:
            st.wait()


def _sc_gather(lo, x, tok):
    return _sc_gather_kernel(lo)(x, tok)


def _ffn_body(t_off, pf_ref, tok_ref, x_ref, w1_ref, w3_ref, w2_ref, wgt_ref,
              out_ref, y_v):
    i = pl.program_id(0)

    @pl.when(i == 0)
    def _():
        out_ref[...] = jnp.zeros_like(out_ref)

    @pl.when(pf_ref[1, t_off + i] == 1)
    def _():
        x = x_ref[...]
        w1 = w1_ref[0]
        w3 = w3_ref[0]
        w2 = w2_ref[0]
        h1 = lax.dot_general(x, w1, (((1,), (1,)), ((), ())),
                             preferred_element_type=jnp.float32)
        h3 = lax.dot_general(x, w3, (((1,), (1,)), ((), ())),
                             preferred_element_type=jnp.float32)
        h = (h1 * lax.logistic(h1)) * h3
        y = lax.dot_general(h, w2, (((1,), (1,)), ((), ())),
                            preferred_element_type=jnp.float32)
        w = wgt_ref[...]
        y_v[...] = y * w[:, None]

        def addrow(r, carry):
            t = tok_ref[(t_off + i) * TILE + r]
            out_ref[pl.ds(t, 1), :] += y_v[pl.ds(r, 1), :]
            return carry

        lax.fori_loop(0, TILE, addrow, 0)


def _ffn_grouped(t_off, pf, tok, x_sorted, w1, w3, w2, wgt):
    grid_spec = pltpu.PrefetchScalarGridSpec(
        num_scalar_prefetch=2,
        grid=(NT_HALF,),
        in_specs=[
            pl.BlockSpec((TILE, H), lambda i, pf, tok: (i, 0)),
            pl.BlockSpec((1, FFN, H),
                         lambda i, pf, tok: (pf[0, t_off + i], 0, 0)),
            pl.BlockSpec((1, FFN, H),
                         lambda i, pf, tok: (pf[0, t_off + i], 0, 0)),
            pl.BlockSpec((1, H, FFN),
                         lambda i, pf, tok: (pf[0, t_off + i], 0, 0)),
            pl.BlockSpec((TILE,), lambda i, pf, tok: (t_off + i,)),
        ],
        out_specs=pl.BlockSpec((T, H), lambda i, pf, tok: (0, 0)),
        scratch_shapes=[pltpu.VMEM((TILE, H), jnp.float32)],
    )
    return pl.pallas_call(
        functools.partial(_ffn_body, t_off),
        grid_spec=grid_spec,
        out_shape=jax.ShapeDtypeStruct((T, H), jnp.float32),
        compiler_params=pltpu.CompilerParams(
            dimension_semantics=("arbitrary",),
            vmem_limit_bytes=100 * 1024 * 1024,
        ),
    )(pf, tok, x_sorted, w1, w3, w2, wgt)


@jax.jit
def kernel(hidden_states, gate_w, w1, w2, w3):
    B, S, Hh = hidden_states.shape
    x = hidden_states.reshape(-1, Hh)
    gate_pad = jnp.zeros((LANES, Hh), jnp.float32).at[:E].set(gate_w)
    (router_logits, pos1, pos2, wn1, wn2, pf) = _router(x, gate_pad)
    tok_sorted, wgt_sorted = _sc_meta(pos1, pos2, wn1, wn2)
    parts = []
    for q in range(NCHUNK):
        xs_q = _sc_gather(q * NP_HALF, x, tok_sorted)
        parts.append(_ffn_grouped(q * NT_HALF, pf, tok_sorted, xs_q,
                                  w1, w3, w2, wgt_sorted))
    final = parts[0] + parts[1] + parts[2] + parts[3]
    return final.reshape(B, S, Hh), router_logits


# speedup vs baseline: 1.0841x; 1.0841x over previous
import functools

import jax
import jax.numpy as jnp
from jax import lax
from jax.experimental import pallas as pl
from jax.experimental.pallas import tpu as pltpu
from jax.experimental.pallas import tpu_sc as plsc

E = 8
TOP_K = 2
LANES = 128
T = 2048
H = 1024
FFN = 2048
TILE = 256
NP = T * TOP_K + E * TILE
NT = NP // TILE
NCHUNK = 2
NT_HALF = NT // NCHUNK
NP_HALF = NP // NCHUNK
NWORK = 32


def _router_body(x_ref, gate_ref, logits_ref, pos1_ref, pos2_ref,
                 wn1_ref, wn2_ref, pf_ref):
    x = x_ref[...]
    gate = gate_ref[...]
    logits = lax.dot_general(x, gate, (((1,), (1,)), ((), ())),
                             preferred_element_type=jnp.float32)
    lane = lax.broadcasted_iota(jnp.int32, (T, LANES), 1)
    neg = jnp.float32(-1e30)
    masked = jnp.where(lane < E, logits, neg)
    m = jnp.max(masked, axis=1, keepdims=True)
    ex = jnp.exp(masked - m)
    p = ex / jnp.sum(ex, axis=1, keepdims=True)
    m1 = jnp.max(p, axis=1, keepdims=True)
    i1 = jnp.min(jnp.where(p == m1, lane, LANES), axis=1, keepdims=True)
    p_rest = jnp.where(lane == i1, jnp.float32(-1.0), p)
    m2 = jnp.max(p_rest, axis=1, keepdims=True)
    i2 = jnp.min(jnp.where(p_rest == m2, lane, LANES), axis=1, keepdims=True)
    denom = m1 + m2
    sel1 = lane == i1
    sel2 = lane == i2
    mask = jnp.where(sel1 | sel2, jnp.float32(1.0), 0.0)
    r_io = lax.broadcasted_iota(jnp.int32, (T, T), 0)
    c_io = lax.broadcasted_iota(jnp.int32, (T, T), 1)
    tril = jnp.where(r_io > c_io, jnp.float32(1.0), 0.0)
    cum = lax.dot_general(tril, mask, (((1,), (0,)), ((), ())),
                          preferred_element_type=jnp.float32)
    counts = jnp.sum(mask, axis=0, keepdims=True)
    padded = jnp.ceil(counts / TILE) * TILE
    ri = lax.broadcasted_iota(jnp.int32, (LANES, LANES), 0)
    ci = lax.broadcasted_iota(jnp.int32, (LANES, LANES), 1)
    upper = jnp.where(ri < ci, jnp.float32(1.0), 0.0)
    offs = lax.dot_general(padded, upper, (((1,), (0,)), ((), ())),
                           preferred_element_type=jnp.float32)
    ends = offs + padded
    pos = offs + cum
    pos1 = jnp.sum(jnp.where(sel1, pos, 0.0), axis=1, keepdims=True)
    pos2 = jnp.sum(jnp.where(sel2, pos, 0.0), axis=1, keepdims=True)
    tstart = (lax.broadcasted_iota(jnp.int32, (LANES, LANES), 0)
              * TILE).astype(jnp.float32)
    lane2 = lax.broadcasted_iota(jnp.int32, (LANES, LANES), 1)
    hit = jnp.where((tstart >= jnp.broadcast_to(ends, (LANES, LANES)))
                    & (lane2 < E), jnp.float32(1.0), 0.0)
    cnt = jnp.sum(hit, axis=1, keepdims=True).astype(jnp.int32)
    logits_ref[...] = logits[:, :E]
    pos1_ref[...] = pos1[:, 0].astype(jnp.int32)
    pos2_ref[...] = pos2[:, 0].astype(jnp.int32)
    wn1_ref[...] = (m1 / denom)[:, 0]
    wn2_ref[...] = (m2 / denom)[:, 0]
    te = jnp.minimum(cnt, E - 1)[:NT, 0]
    valid = jnp.where(cnt < E, 1, 0).astype(jnp.int32)[:NT, 0]
    pf_ref[0, :] = te
    pf_ref[1, :] = valid


def _router(x, gate_pad):
    return pl.pallas_call(
        _router_body,
        out_shape=(
            jax.ShapeDtypeStruct((T, E), jnp.float32),
            jax.ShapeDtypeStruct((T,), jnp.int32),
            jax.ShapeDtypeStruct((T,), jnp.int32),
            jax.ShapeDtypeStruct((T,), jnp.float32),
            jax.ShapeDtypeStruct((T,), jnp.float32),
            jax.ShapeDtypeStruct((2, NT), jnp.int32),
        ),
    )(x, gate_pad)


def _sc_mesh():
    return plsc.VectorSubcoreMesh(core_axis_name="c", subcore_axis_name="s",
                                  num_cores=2, num_subcores=16)


@functools.lru_cache(maxsize=None)
def _sc_meta_kernel():
    return functools.partial(
        pl.kernel,
        mesh=_sc_mesh(),
        out_type=(
            jax.ShapeDtypeStruct((NP,), jnp.int32),
            jax.ShapeDtypeStruct((NP,), jnp.float32),
        ),
        scratch_types=[
            pltpu.VMEM((T,), jnp.int32),
            pltpu.VMEM((T,), jnp.int32),
            pltpu.VMEM((T,), jnp.float32),
            pltpu.VMEM((T,), jnp.float32),
            pltpu.VMEM((NP,), jnp.int32),
            pltpu.VMEM((NP,), jnp.float32),
        ],
        compiler_params=pltpu.CompilerParams(needs_layout_passes=False),
    )(_sc_meta_body)


def _sc_meta_body(p1_hbm, p2_hbm, a1_hbm, a2_hbm, tok_hbm, wgt_hbm,
                  p1_v, p2_v, a1_v, a2_v, tok_v, wgt_v):
    wid = lax.axis_index("s") * 2 + lax.axis_index("c")

    @pl.when(wid == 0)
    def _():
        pltpu.sync_copy(p1_hbm, p1_v)
        pltpu.sync_copy(p2_hbm, p2_v)
        pltpu.sync_copy(a1_hbm, a1_v)
        pltpu.sync_copy(a2_hbm, a2_v)

        def init(i, carry):
            tok_v[pl.ds(i * 16, 16)] = jnp.zeros((16,), jnp.int32)
            wgt_v[pl.ds(i * 16, 16)] = jnp.zeros((16,), jnp.float32)
            return carry

        lax.fori_loop(0, NP // 16, init, 0)

        def scat(i, carry):
            sl = pl.ds(i * 16, 16)
            tvec = lax.iota(jnp.int32, 16) + i * 16
            plsc.store_scatter(tok_v, [p1_v[sl]], tvec)
            plsc.store_scatter(wgt_v, [p1_v[sl]], a1_v[sl])
            plsc.store_scatter(tok_v, [p2_v[sl]], tvec)
            plsc.store_scatter(wgt_v, [p2_v[sl]], a2_v[sl])
            return carry

        lax.fori_loop(0, T // 16, scat, 0)
        pltpu.sync_copy(tok_v, tok_hbm)
        pltpu.sync_copy(wgt_v, wgt_hbm)


def _sc_meta(p1, p2, a1, a2):
    return _sc_meta_kernel()(p1, p2, a1, a2)


_SLOTS_PER = NP_HALF // NWORK
_GCH = _SLOTS_PER // 2


@functools.lru_cache(maxsize=None)
def _sc_gather_kernel(lo):
    body = functools.partial(_sc_gather_body, lo)
    return functools.partial(
        pl.kernel,
        mesh=_sc_mesh(),
        out_type=jax.ShapeDtypeStruct((NP_HALF, H), jnp.float32),
        scratch_types=[
            pltpu.VMEM((_SLOTS_PER,), jnp.int32),
            pltpu.VMEM((_GCH, H), jnp.float32),
            pltpu.VMEM((_GCH, H), jnp.float32),
            pltpu.SemaphoreType.DMA,
            pltpu.SemaphoreType.DMA,
            pltpu.SemaphoreType.DMA,
            pltpu.SemaphoreType.DMA,
        ],
        compiler_params=pltpu.CompilerParams(needs_layout_passes=False),
    )(body)


def _sc_gather_body(lo, x_hbm, tok_hbm, xs_hbm,
                    idx_v, rows_a, rows_b, sem_ga, sem_gb, sem_sa, sem_sb):
    wid = lax.axis_index("s") * 2 + lax.axis_index("c")
    base = wid * _SLOTS_PER
    pltpu.sync_copy(tok_hbm.at[pl.ds(lo + base, _SLOTS_PER)], idx_v)
    bufs = (rows_a, rows_b)
    gsems = (sem_ga, sem_gb)
    ssems = (sem_sa, sem_sb)
    stores = [None, None]
    for c in range(_SLOTS_PER // _GCH):
        b = c % 2
        if stores[b] is not None:
            stores[b].wait()
        pltpu.async_copy(
            x_hbm.at[idx_v.at[pl.ds(c * _GCH, _GCH)]], bufs[b], gsems[b]
        ).wait()
        stores[b] = pltpu.async_copy(
            bufs[b], xs_hbm.at[pl.ds(base + c * _GCH, _GCH)], ssems[b]
        )
    for st in stores:
        if st is not None:
            st.wait()


def _sc_gather(lo, x, tok):
    return _sc_gather_kernel(lo)(x, tok)


def _ffn_body(t_off, pf_ref, tok_ref, x_ref, w1_ref, w3_ref, w2_ref, wgt_ref,
              out_ref, y_v):
    i = pl.program_id(0)

    @pl.when(i == 0)
    def _():
        out_ref[...] = jnp.zeros_like(out_ref)

    @pl.when(pf_ref[1, t_off + i] == 1)
    def _():
        x = x_ref[...]
        w1 = w1_ref[0]
        w3 = w3_ref[0]
        w2 = w2_ref[0]
        h1 = lax.dot_general(x, w1, (((1,), (1,)), ((), ())),
                             preferred_element_type=jnp.float32)
        h3 = lax.dot_general(x, w3, (((1,), (1,)), ((), ())),
                             preferred_element_type=jnp.float32)
        h = (h1 * lax.logistic(h1)) * h3
        y = lax.dot_general(h, w2, (((1,), (1,)), ((), ())),
                            preferred_element_type=jnp.float32)
        w = wgt_ref[...]
        y_v[...] = y * w[:, None]

        def addrow(r, carry):
            t = tok_ref[(t_off + i) * TILE + r]
            out_ref[pl.ds(t, 1), :] += y_v[pl.ds(r, 1), :]
            return carry

        lax.fori_loop(0, TILE, addrow, 0)


def _ffn_grouped(t_off, pf, tok, x_sorted, w1, w3, w2, wgt):
    grid_spec = pltpu.PrefetchScalarGridSpec(
        num_scalar_prefetch=2,
        grid=(NT_HALF,),
        in_specs=[
            pl.BlockSpec((TILE, H), lambda i, pf, tok: (i, 0)),
            pl.BlockSpec((1, FFN, H),
                         lambda i, pf, tok: (pf[0, t_off + i], 0, 0)),
            pl.BlockSpec((1, FFN, H),
                         lambda i, pf, tok: (pf[0, t_off + i], 0, 0)),
            pl.BlockSpec((1, H, FFN),
                         lambda i, pf, tok: (pf[0, t_off + i], 0, 0)),
            pl.BlockSpec((TILE,), lambda i, pf, tok: (t_off + i,)),
        ],
        out_specs=pl.BlockSpec((T, H), lambda i, pf, tok: (0, 0)),
        scratch_shapes=[pltpu.VMEM((TILE, H), jnp.float32)],
    )
    return pl.pallas_call(
        functools.partial(_ffn_body, t_off),
        grid_spec=grid_spec,
        out_shape=jax.ShapeDtypeStruct((T, H), jnp.float32),
        compiler_params=pltpu.CompilerParams(
            dimension_semantics=("arbitrary",),
            vmem_limit_bytes=100 * 1024 * 1024,
        ),
    )(pf, tok, x_sorted, w1, w3, w2, wgt)


@jax.jit
def kernel(hidden_states, gate_w, w1, w2, w3):
    B, S, Hh = hidden_states.shape
    x = hidden_states.reshape(-1, Hh)
    gate_pad = jnp.zeros((LANES, Hh), jnp.float32).at[:E].set(gate_w)
    (router_logits, pos1, pos2, wn1, wn2, pf) = _router(x, gate_pad)
    tok_sorted, wgt_sorted = _sc_meta(pos1, pos2, wn1, wn2)
    parts = []
    for q in range(NCHUNK):
        xs_q = _sc_gather(q * NP_HALF, x, tok_sorted)
        parts.append(_ffn_grouped(q * NT_HALF, pf, tok_sorted, xs_q,
                                  w1, w3, w2, wgt_sorted))
    final = sum(parts[1:], parts[0])
    return final.reshape(B, S, Hh), router_logits


# speedup vs baseline: 1.0953x; 1.0103x over previous
import functools

import jax
import jax.numpy as jnp
from jax import lax
from jax.experimental import pallas as pl
from jax.experimental.pallas import tpu as pltpu
from jax.experimental.pallas import tpu_sc as plsc

E = 8
TOP_K = 2
LANES = 128
T = 2048
H = 1024
FFN = 2048
TILE = 256
NP = T * TOP_K + E * TILE
NT = NP // TILE
NCHUNK = 2
NT_HALF = NT // NCHUNK
NP_HALF = NP // NCHUNK
NWORK = 32


def _router_body(x_ref, gate_ref, logits_ref, pos1_ref, pos2_ref,
                 wn1_ref, wn2_ref, pf_ref):
    x = x_ref[...]
    gate = gate_ref[...]
    logits = lax.dot_general(x, gate, (((1,), (1,)), ((), ())),
                             preferred_element_type=jnp.float32)
    lane = lax.broadcasted_iota(jnp.int32, (T, LANES), 1)
    neg = jnp.float32(-1e30)
    masked = jnp.where(lane < E, logits, neg)
    m = jnp.max(masked, axis=1, keepdims=True)
    ex = jnp.exp(masked - m)
    p = ex / jnp.sum(ex, axis=1, keepdims=True)
    m1 = jnp.max(p, axis=1, keepdims=True)
    i1 = jnp.min(jnp.where(p == m1, lane, LANES), axis=1, keepdims=True)
    p_rest = jnp.where(lane == i1, jnp.float32(-1.0), p)
    m2 = jnp.max(p_rest, axis=1, keepdims=True)
    i2 = jnp.min(jnp.where(p_rest == m2, lane, LANES), axis=1, keepdims=True)
    denom = m1 + m2
    sel1 = lane == i1
    sel2 = lane == i2
    mask = jnp.where(sel1 | sel2, jnp.float32(1.0), 0.0)
    r_io = lax.broadcasted_iota(jnp.int32, (T, T), 0)
    c_io = lax.broadcasted_iota(jnp.int32, (T, T), 1)
    tril = jnp.where(r_io > c_io, jnp.float32(1.0), 0.0)
    cum = lax.dot_general(tril, mask, (((1,), (0,)), ((), ())),
                          preferred_element_type=jnp.float32)
    counts = jnp.sum(mask, axis=0, keepdims=True)
    padded = jnp.ceil(counts / TILE) * TILE
    ri = lax.broadcasted_iota(jnp.int32, (LANES, LANES), 0)
    ci = lax.broadcasted_iota(jnp.int32, (LANES, LANES), 1)
    upper = jnp.where(ri < ci, jnp.float32(1.0), 0.0)
    offs = lax.dot_general(padded, upper, (((1,), (0,)), ((), ())),
                           preferred_element_type=jnp.float32)
    ends = offs + padded
    pos = offs + cum
    pos1 = jnp.sum(jnp.where(sel1, pos, 0.0), axis=1, keepdims=True)
    pos2 = jnp.sum(jnp.where(sel2, pos, 0.0), axis=1, keepdims=True)
    tstart = (lax.broadcasted_iota(jnp.int32, (LANES, LANES), 0)
              * TILE).astype(jnp.float32)
    lane2 = lax.broadcasted_iota(jnp.int32, (LANES, LANES), 1)
    hit = jnp.where((tstart >= jnp.broadcast_to(ends, (LANES, LANES)))
                    & (lane2 < E), jnp.float32(1.0), 0.0)
    cnt = jnp.sum(hit, axis=1, keepdims=True).astype(jnp.int32)
    logits_ref[...] = logits[:, :E]
    pos1_ref[...] = pos1[:, 0].astype(jnp.int32)
    pos2_ref[...] = pos2[:, 0].astype(jnp.int32)
    wn1_ref[...] = (m1 / denom)[:, 0]
    wn2_ref[...] = (m2 / denom)[:, 0]
    te = jnp.minimum(cnt, E - 1)[:NT, 0]
    valid = jnp.where(cnt < E, 1, 0).astype(jnp.int32)[:NT, 0]
    pf_ref[0, :] = te
    pf_ref[1, :] = valid


def _router(x, gate_pad):
    return pl.pallas_call(
        _router_body,
        out_shape=(
            jax.ShapeDtypeStruct((T, E), jnp.float32),
            jax.ShapeDtypeStruct((T,), jnp.int32),
            jax.ShapeDtypeStruct((T,), jnp.int32),
            jax.ShapeDtypeStruct((T,), jnp.float32),
            jax.ShapeDtypeStruct((T,), jnp.float32),
            jax.ShapeDtypeStruct((2, NT), jnp.int32),
        ),
    )(x, gate_pad)


def _sc_mesh():
    return plsc.VectorSubcoreMesh(core_axis_name="c", subcore_axis_name="s",
                                  num_cores=2, num_subcores=16)


@functools.lru_cache(maxsize=None)
def _sc_meta_kernel():
    return functools.partial(
        pl.kernel,
        mesh=_sc_mesh(),
        out_type=(
            jax.ShapeDtypeStruct((NP,), jnp.int32),
            jax.ShapeDtypeStruct((NP,), jnp.float32),
        ),
        scratch_types=[
            pltpu.VMEM((T,), jnp.int32),
            pltpu.VMEM((T,), jnp.int32),
            pltpu.VMEM((T,), jnp.float32),
            pltpu.VMEM((T,), jnp.float32),
            pltpu.VMEM((NP,), jnp.int32),
            pltpu.VMEM((NP,), jnp.float32),
        ],
        compiler_params=pltpu.CompilerParams(needs_layout_passes=False),
    )(_sc_meta_body)


def _sc_meta_body(p1_hbm, p2_hbm, a1_hbm, a2_hbm, tok_hbm, wgt_hbm,
                  p1_v, p2_v, a1_v, a2_v, tok_v, wgt_v):
    wid = lax.axis_index("s") * 2 + lax.axis_index("c")

    @pl.when(wid == 0)
    def _():
        pltpu.sync_copy(p1_hbm, p1_v)
        pltpu.sync_copy(p2_hbm, p2_v)
        pltpu.sync_copy(a1_hbm, a1_v)
        pltpu.sync_copy(a2_hbm, a2_v)

        def init(i, carry):
            tok_v[pl.ds(i * 16, 16)] = jnp.zeros((16,), jnp.int32)
            wgt_v[pl.ds(i * 16, 16)] = jnp.zeros((16,), jnp.float32)
            return carry

        lax.fori_loop(0, NP // 16, init, 0)

        def scat(i, carry):
            sl = pl.ds(i * 16, 16)
            tvec = lax.iota(jnp.int32, 16) + i * 16
            plsc.store_scatter(tok_v, [p1_v[sl]], tvec)
            plsc.store_scatter(wgt_v, [p1_v[sl]], a1_v[sl])
            plsc.store_scatter(tok_v, [p2_v[sl]], tvec)
            plsc.store_scatter(wgt_v, [p2_v[sl]], a2_v[sl])
            return carry

        lax.fori_loop(0, T // 16, scat, 0)
        pltpu.sync_copy(tok_v, tok_hbm)
        pltpu.sync_copy(wgt_v, wgt_hbm)


def _sc_meta(p1, p2, a1, a2):
    return _sc_meta_kernel()(p1, p2, a1, a2)


_SLOTS_PER = NP_HALF // NWORK
_GCH = _SLOTS_PER


@functools.lru_cache(maxsize=None)
def _sc_gather_kernel(lo):
    body = functools.partial(_sc_gather_body, lo)
    return functools.partial(
        pl.kernel,
        mesh=_sc_mesh(),
        out_type=jax.ShapeDtypeStruct((NP_HALF, H), jnp.float32),
        scratch_types=[
            pltpu.VMEM((_SLOTS_PER,), jnp.int32),
            pltpu.VMEM((_GCH, H), jnp.float32),
            pltpu.SemaphoreType.DMA,
        ],
        compiler_params=pltpu.CompilerParams(needs_layout_passes=False),
    )(body)


def _sc_gather_body(lo, x_hbm, tok_hbm, xs_hbm, idx_v, rows_v, sem):
    wid = lax.axis_index("s") * 2 + lax.axis_index("c")
    base = wid * _SLOTS_PER
    pltpu.sync_copy(tok_hbm.at[pl.ds(lo + base, _SLOTS_PER)], idx_v)
    pltpu.async_copy(x_hbm.at[idx_v], rows_v, sem).wait()
    pltpu.sync_copy(rows_v, xs_hbm.at[pl.ds(base, _SLOTS_PER)])


def _sc_gather(lo, x, tok):
    return _sc_gather_kernel(lo)(x, tok)


def _ffn_body(t_off, pf_ref, tok_ref, x_ref, w1_ref, w3_ref, w2_ref, wgt_ref,
              out_ref, y_v):
    i = pl.program_id(0)

    @pl.when(i == 0)
    def _():
        out_ref[...] = jnp.zeros_like(out_ref)

    @pl.when(pf_ref[1, t_off + i] == 1)
    def _():
        x = x_ref[...]
        w1 = w1_ref[0]
        w3 = w3_ref[0]
        w2 = w2_ref[0]
        h1 = lax.dot_general(x, w1, (((1,), (1,)), ((), ())),
                             preferred_element_type=jnp.float32)
        h3 = lax.dot_general(x, w3, (((1,), (1,)), ((), ())),
                             preferred_element_type=jnp.float32)
        h = (h1 * lax.logistic(h1)) * h3
        y = lax.dot_general(h, w2, (((1,), (1,)), ((), ())),
                            preferred_element_type=jnp.float32)
        w = wgt_ref[...]
        y_v[...] = y * w[:, None]

        def addrow(r, carry):
            t = tok_ref[(t_off + i) * TILE + r]
            out_ref[pl.ds(t, 1), :] += y_v[pl.ds(r, 1), :]
            return carry

        lax.fori_loop(0, TILE, addrow, 0)


def _ffn_grouped(t_off, pf, tok, x_sorted, w1, w3, w2, wgt):
    grid_spec = pltpu.PrefetchScalarGridSpec(
        num_scalar_prefetch=2,
        grid=(NT_HALF,),
        in_specs=[
            pl.BlockSpec((TILE, H), lambda i, pf, tok: (i, 0)),
            pl.BlockSpec((1, FFN, H),
                         lambda i, pf, tok: (pf[0, t_off + i], 0, 0)),
            pl.BlockSpec((1, FFN, H),
                         lambda i, pf, tok: (pf[0, t_off + i], 0, 0)),
            pl.BlockSpec((1, H, FFN),
                         lambda i, pf, tok: (pf[0, t_off + i], 0, 0)),
            pl.BlockSpec((TILE,), lambda i, pf, tok: (t_off + i,)),
        ],
        out_specs=pl.BlockSpec((T, H), lambda i, pf, tok: (0, 0)),
        scratch_shapes=[pltpu.VMEM((TILE, H), jnp.float32)],
    )
    return pl.pallas_call(
        functools.partial(_ffn_body, t_off),
        grid_spec=grid_spec,
        out_shape=jax.ShapeDtypeStruct((T, H), jnp.float32),
        compiler_params=pltpu.CompilerParams(
            dimension_semantics=("arbitrary",),
            vmem_limit_bytes=100 * 1024 * 1024,
        ),
    )(pf, tok, x_sorted, w1, w3, w2, wgt)


@jax.jit
def kernel(hidden_states, gate_w, w1, w2, w3):
    B, S, Hh = hidden_states.shape
    x = hidden_states.reshape(-1, Hh)
    gate_pad = jnp.zeros((LANES, Hh), jnp.float32).at[:E].set(gate_w)
    (router_logits, pos1, pos2, wn1, wn2, pf) = _router(x, gate_pad)
    tok_sorted, wgt_sorted = _sc_meta(pos1, pos2, wn1, wn2)
    parts = []
    for q in range(NCHUNK):
        xs_q = _sc_gather(q * NP_HALF, x, tok_sorted)
        parts.append(_ffn_grouped(q * NT_HALF, pf, tok_sorted, xs_q,
                                  w1, w3, w2, wgt_sorted))
    final = sum(parts[1:], parts[0])
    return final.reshape(B, S, Hh), router_logits
